# baseline (speedup 1.0000x reference)
import jax
import jax.numpy as jnp
from jax.experimental import pallas as pl
from jax.experimental.pallas import tpu as pltpu


D = 4096
BLK = 512
CHUNK = 512
GBLK = 256
NB = D // BLK
NGB = D // GBLK
ITERS = 4
G_A = NB
G_B = G_A + ITERS
G_TOT = G_B + NGB
EPS = 1e-6


def _normalized(x):
    return x * jax.lax.rsqrt(jnp.sum(x * x))


def _mega_kernel(a_ref, g_ref, m_ref, mb, rs_row, cs_row, v_row, vt_row):
    g = pl.program_id(0)

    @pl.when(g < G_A)
    def _phase_a():
        a = a_ref[...]
        sq = a * a
        m_ref[...] = sq
        base = pl.multiple_of(g * BLK, BLK)
        mb[pl.ds(base, BLK), :] = sq.astype(jnp.float8_e4m3fn)
        rs_row[:, pl.ds(base, BLK)] = (
            jnp.sum(sq, axis=1, keepdims=True).reshape(1, BLK))
        part = jnp.sum(sq, axis=0, keepdims=True)

        @pl.when(g == 0)
        def _init():
            cs_row[...] = part

        @pl.when(g != 0)
        def _acc():
            cs_row[...] += part

    @pl.when(g == G_A)
    def _iter_init():
        eps_d = jnp.float32(EPS) * D
        v_row[...] = _normalized(rs_row[...] + eps_d)
        vt_row[...] = _normalized(cs_row[...] + eps_d)

    @pl.when((g >= G_A) & (g < G_B))
    def _phase_b():
        vrow = v_row[...]
        vtrow = vt_row[...]
        sv = jnp.sum(vrow) * jnp.float32(EPS)
        svt = jnp.sum(vtrow) * jnp.float32(EPS)
        vb = vrow.astype(jnp.float8_e4m3fn)
        vtb = vtrow.astype(jnp.float8_e4m3fn)
        y_parts = []
        yt_parts = []
        for c in range(0, D, CHUNK):
            m_chunk = mb[c:c + CHUNK, :]
            y_parts.append(jax.lax.dot_general(
                vb, m_chunk, (((1,), (1,)), ((), ())),
                preferred_element_type=jnp.float32))
            yt_parts.append(jax.lax.dot_general(
                vtb[:, c:c + CHUNK], m_chunk, (((1,), (0,)), ((), ())),
                preferred_element_type=jnp.float32))
        while len(yt_parts) > 1:
            yt_parts = [a + b for a, b in zip(yt_parts[::2], yt_parts[1::2])]
        yt = yt_parts[0]
        y = jnp.concatenate(y_parts, axis=1)
        v_row[...] = _normalized(y + sv)
        vt_row[...] = _normalized(yt + svt)

    @pl.when(g >= G_B)
    def _phase_c():
        t = g - G_B

        @pl.when(g == G_B)
        def _scale():
            inv_dot = jnp.float32(1.0) / jnp.sum(v_row[...] * vt_row[...])
            rs_row[...] = vt_row[...] * inv_dot

        vts_blk = rs_row[:, pl.ds(pl.multiple_of(t * GBLK, GBLK), GBLK)]
        g_ref[...] = vts_blk.reshape(GBLK, 1) * v_row[...]
        lanes = pl.ds(pl.multiple_of(t * GBLK, GBLK), GBLK)
        r_ids = jax.lax.broadcasted_iota(jnp.int32, (GBLK, GBLK), 0)
        c_ids = jax.lax.broadcasted_iota(jnp.int32, (GBLK, GBLK), 1)
        eye = jnp.where(r_ids == c_ids, jnp.float32(100.0), jnp.float32(0.0))
        g_ref[:, lanes] += eye


def kernel(adj_mtx):
    f32 = jnp.float32
    last = NB - 1

    gradient, matrix = pl.pallas_call(
        _mega_kernel,
        grid=(G_TOT,),
        compiler_params=pltpu.CompilerParams(
            vmem_limit_bytes=64 * 1024 * 1024),
        in_specs=[
            pl.BlockSpec((BLK, D), lambda i: (jnp.minimum(i, last), 0)),
        ],
        out_specs=[
            pl.BlockSpec((GBLK, D), lambda i: (jnp.maximum(i - G_B, 0), 0)),
            pl.BlockSpec((BLK, D), lambda i: (jnp.minimum(i, last), 0)),
        ],
        out_shape=[
            jax.ShapeDtypeStruct((D, D), f32),
            jax.ShapeDtypeStruct((D, D), f32),
        ],
        scratch_shapes=[
            pltpu.VMEM((D, D), jnp.float8_e4m3fn),
            pltpu.VMEM((1, D), f32),
            pltpu.VMEM((1, D), f32),
            pltpu.VMEM((1, D), f32),
            pltpu.VMEM((1, D), f32),
        ],
    )(adj_mtx)

    return (gradient, matrix)


# speedup vs baseline: 1.0034x; 1.0034x over previous
import jax
import jax.numpy as jnp
from jax.experimental import pallas as pl
from jax.experimental.pallas import tpu as pltpu


D = 4096
BLK = 512
CHUNK = 1024
GBLK = 256
NB = D // BLK
NGB = D // GBLK
ITERS = 4
G_A = NB
G_B = G_A + ITERS
G_TOT = G_B + NGB
EPS = 1e-6


def _normalized(x):
    return x * jax.lax.rsqrt(jnp.sum(x * x))


def _mega_kernel(a_ref, g_ref, m_ref, mb, rs_row, cs_row, v_row, vt_row):
    g = pl.program_id(0)

    @pl.when(g < G_A)
    def _phase_a():
        a = a_ref[...]
        sq = a * a
        m_ref[...] = sq
        base = pl.multiple_of(g * BLK, BLK)
        mb[pl.ds(base, BLK), :] = sq.astype(jnp.float8_e4m3fn)
        rs_row[:, pl.ds(base, BLK)] = (
            jnp.sum(sq, axis=1, keepdims=True).reshape(1, BLK))
        part = jnp.sum(sq, axis=0, keepdims=True)

        @pl.when(g == 0)
        def _init():
            cs_row[...] = part

        @pl.when(g != 0)
        def _acc():
            cs_row[...] += part

    @pl.when(g == G_A)
    def _iter_init():
        eps_d = jnp.float32(EPS) * D
        v_row[...] = _normalized(rs_row[...] + eps_d)
        vt_row[...] = _normalized(cs_row[...] + eps_d)

    @pl.when((g >= G_A) & (g < G_B))
    def _phase_b():
        vrow = v_row[...]
        vtrow = vt_row[...]
        sv = jnp.sum(vrow) * jnp.float32(EPS)
        svt = jnp.sum(vtrow) * jnp.float32(EPS)
        vb = vrow.astype(jnp.float8_e4m3fn)
        vtb = vtrow.astype(jnp.float8_e4m3fn)
        y_parts = []
        yt_parts = []
        for c in range(0, D, CHUNK):
            m_chunk = mb[c:c + CHUNK, :]
            y_parts.append(jax.lax.dot_general(
                vb, m_chunk, (((1,), (1,)), ((), ())),
                preferred_element_type=jnp.float32))
            yt_parts.append(jax.lax.dot_general(
                vtb[:, c:c + CHUNK], m_chunk, (((1,), (0,)), ((), ())),
                preferred_element_type=jnp.float32))
        while len(yt_parts) > 1:
            yt_parts = [a + b for a, b in zip(yt_parts[::2], yt_parts[1::2])]
        yt = yt_parts[0]
        y = jnp.concatenate(y_parts, axis=1)
        v_row[...] = _normalized(y + sv)
        vt_row[...] = _normalized(yt + svt)

    @pl.when(g >= G_B)
    def _phase_c():
        t = g - G_B

        @pl.when(g == G_B)
        def _scale():
            inv_dot = jnp.float32(1.0) / jnp.sum(v_row[...] * vt_row[...])
            rs_row[...] = vt_row[...] * inv_dot

        vts_blk = rs_row[:, pl.ds(pl.multiple_of(t * GBLK, GBLK), GBLK)]
        g_ref[...] = vts_blk.reshape(GBLK, 1) * v_row[...]
        lanes = pl.ds(pl.multiple_of(t * GBLK, GBLK), GBLK)
        r_ids = jax.lax.broadcasted_iota(jnp.int32, (GBLK, GBLK), 0)
        c_ids = jax.lax.broadcasted_iota(jnp.int32, (GBLK, GBLK), 1)
        eye = jnp.where(r_ids == c_ids, jnp.float32(100.0), jnp.float32(0.0))
        g_ref[:, lanes] += eye


def kernel(adj_mtx):
    f32 = jnp.float32
    last = NB - 1

    gradient, matrix = pl.pallas_call(
        _mega_kernel,
        grid=(G_TOT,),
        compiler_params=pltpu.CompilerParams(
            vmem_limit_bytes=64 * 1024 * 1024),
        in_specs=[
            pl.BlockSpec((BLK, D), lambda i: (jnp.minimum(i, last), 0)),
        ],
        out_specs=[
            pl.BlockSpec((GBLK, D), lambda i: (jnp.maximum(i - G_B, 0), 0)),
            pl.BlockSpec((BLK, D), lambda i: (jnp.minimum(i, last), 0)),
        ],
        out_shape=[
            jax.ShapeDtypeStruct((D, D), f32),
            jax.ShapeDtypeStruct((D, D), f32),
        ],
        scratch_shapes=[
            pltpu.VMEM((D, D), jnp.float8_e4m3fn),
            pltpu.VMEM((1, D), f32),
            pltpu.VMEM((1, D), f32),
            pltpu.VMEM((1, D), f32),
            pltpu.VMEM((1, D), f32),
        ],
    )(adj_mtx)

    return (gradient, matrix)
